# Initial kernel scaffold; baseline (speedup 1.0000x reference)
import jax
import jax.numpy as jnp
from jax.experimental import pallas as pl


def kernel(input, table):
    raise NotImplementedError("write your pallas kernel here")


# speedup vs baseline: 1.0230x; 1.0230x over previous
import functools

import jax
import jax.numpy as jnp
from jax import lax
from jax.experimental import pallas as pl
from jax.experimental.pallas import tpu as pltpu
from jax.experimental.pallas import tpu_sc as plsc

_NC = 2
_NS = 16
_NW = _NC * _NS
_CHUNK = 128


def _build(B, V, D):
    b_per_w = B // _NW
    n_chunks = b_per_w // _CHUNK
    mesh = plsc.VectorSubcoreMesh(core_axis_name="c", subcore_axis_name="s")

    @functools.partial(
        pl.kernel,
        mesh=mesh,
        out_type=jax.ShapeDtypeStruct((B, D), jnp.float32),
        compiler_params=pltpu.CompilerParams(use_tc_tiling_on_sc=False),
        scratch_types=[
            pltpu.VMEM((n_chunks, _CHUNK), jnp.int32),
            pltpu.VMEM((_CHUNK, D), jnp.float32),
            pltpu.SemaphoreType.DMA,
        ],
    )
    def k(table_hbm, idx_hbm, out_hbm, idx_v, rows_v, sem):
        wid = lax.axis_index("s") * _NC + lax.axis_index("c")
        row_base = wid * n_chunks
        pltpu.sync_copy(idx_hbm.at[pl.ds(row_base, n_chunks)], idx_v)

        def body(j, carry):
            pltpu.async_copy(table_hbm.at[idx_v.at[j]], rows_v, sem).wait()
            pltpu.sync_copy(
                rows_v, out_hbm.at[pl.ds((row_base + j) * _CHUNK, _CHUNK)]
            )
            return carry

        lax.fori_loop(0, n_chunks, body, 0)

    return k


def kernel(input, table):
    B0, B1 = input.shape
    B = B0 * B1
    V, D = table.shape
    idx2d = input.reshape(B // _CHUNK, _CHUNK).astype(jnp.int32)
    out = _build(B, V, D)(table, idx2d)
    return out.reshape(B0, B1, D)


# speedup vs baseline: 1.0897x; 1.0652x over previous
import functools

import jax
import jax.numpy as jnp
from jax import lax
from jax.experimental import pallas as pl
from jax.experimental.pallas import tpu as pltpu
from jax.experimental.pallas import tpu_sc as plsc

_NC = 2
_NS = 16
_NW = _NC * _NS
_CHUNK = 512


def _build(B, V, D):
    b_per_w = B // _NW
    n_chunks = b_per_w // _CHUNK
    mesh = plsc.VectorSubcoreMesh(core_axis_name="c", subcore_axis_name="s")

    @functools.partial(
        pl.kernel,
        mesh=mesh,
        out_type=jax.ShapeDtypeStruct((B, D), jnp.float32),
        compiler_params=pltpu.CompilerParams(use_tc_tiling_on_sc=False),
        scratch_types=[
            pltpu.VMEM((n_chunks, _CHUNK), jnp.int32),
            pltpu.VMEM((_CHUNK, D), jnp.float32),
            pltpu.SemaphoreType.DMA,
        ],
    )
    def k(table_hbm, idx_hbm, out_hbm, idx_v, rows_v, sem):
        wid = lax.axis_index("s") * _NC + lax.axis_index("c")
        row_base = wid * n_chunks
        pltpu.sync_copy(idx_hbm.at[pl.ds(row_base, n_chunks)], idx_v)

        def body(j, carry):
            pltpu.async_copy(table_hbm.at[idx_v.at[j]], rows_v, sem).wait()
            pltpu.sync_copy(
                rows_v, out_hbm.at[pl.ds((row_base + j) * _CHUNK, _CHUNK)]
            )
            return carry

        lax.fori_loop(0, n_chunks, body, 0)

    return k


def kernel(input, table):
    B0, B1 = input.shape
    B = B0 * B1
    V, D = table.shape
    idx2d = input.reshape(B // _CHUNK, _CHUNK).astype(jnp.int32)
    out = _build(B, V, D)(table, idx2d)
    return out.reshape(B0, B1, D)


# speedup vs baseline: 1.7919x; 1.6443x over previous
import functools

import jax
import jax.numpy as jnp
from jax import lax
from jax.experimental import pallas as pl
from jax.experimental.pallas import tpu as pltpu
from jax.experimental.pallas import tpu_sc as plsc

_NC = 2
_NS = 16
_NW = _NC * _NS
_BLK = 16


def _build(B0, B1, V, D):
    rows_per_w = B0 // _NW
    n_blocks = rows_per_w // _BLK
    mesh = plsc.VectorSubcoreMesh(core_axis_name="c", subcore_axis_name="s")

    @functools.partial(
        pl.kernel,
        mesh=mesh,
        out_type=jax.ShapeDtypeStruct((B0, B1, D), jnp.float32),
        compiler_params=pltpu.CompilerParams(use_tc_tiling_on_sc=False),
        scratch_types=[
            pltpu.VMEM((rows_per_w, B1), jnp.int32),
            pltpu.VMEM((_BLK, B1, D), jnp.float32),
            pltpu.VMEM((_BLK, B1, D), jnp.float32),
            pltpu.SemaphoreType.DMA,
            pltpu.SemaphoreType.DMA,
            pltpu.SemaphoreType.DMA,
            pltpu.SemaphoreType.DMA,
        ],
    )
    def k(idx_hbm, table_hbm, out_hbm, idx_v, buf_a, buf_b, gsa, gsb, wsa, wsb):
        wid = lax.axis_index("s") * _NC + lax.axis_index("c")
        b0 = wid * rows_per_w
        pltpu.sync_copy(idx_hbm.at[pl.ds(b0, rows_per_w)], idx_v)

        def fire(blk, buf, sem):
            handles = []
            for i in range(_BLK):
                handles.append(
                    pltpu.async_copy(
                        table_hbm.at[idx_v.at[blk * _BLK + i]], buf.at[i], sem
                    )
                )
            return handles

        def drain(handles):
            for h in handles:
                h.wait()

        def write(blk, buf, sem):
            return pltpu.async_copy(
                buf, out_hbm.at[pl.ds(b0 + blk * _BLK, _BLK)], sem
            )

        drain(fire(0, buf_a, gsa))
        wa = write(0, buf_a, wsa)
        drain(fire(1, buf_b, gsb))
        wb = write(1, buf_b, wsb)

        def body(i, carry):
            blk = 2 * i + 2
            wa.wait()
            drain(fire(blk, buf_a, gsa))
            write(blk, buf_a, wsa)
            wb.wait()
            drain(fire(blk + 1, buf_b, gsb))
            write(blk + 1, buf_b, wsb)
            return carry

        lax.fori_loop(0, (n_blocks - 2) // 2, body, 0)
        wa.wait()
        wb.wait()

    return k


def kernel(input, table):
    B0, B1 = input.shape
    V, D = table.shape
    return _build(B0, B1, V, D)(input.astype(jnp.int32), table)


# speedup vs baseline: 1.8680x; 1.0425x over previous
import functools

import jax
import jax.numpy as jnp
from jax import lax
from jax.experimental import pallas as pl
from jax.experimental.pallas import tpu as pltpu
from jax.experimental.pallas import tpu_sc as plsc

_NC = 2
_NS = 16
_NW = _NC * _NS


def _build(B0, B1, V, D):
    bw = B0 // _NW
    mesh = plsc.VectorSubcoreMesh(core_axis_name="c", subcore_axis_name="s")

    @functools.partial(
        pl.kernel,
        mesh=mesh,
        out_type=jax.ShapeDtypeStruct((B1, B0, D), jnp.float32),
        compiler_params=pltpu.CompilerParams(use_tc_tiling_on_sc=False),
        scratch_types=[
            pltpu.VMEM((bw,), jnp.int32),
            pltpu.VMEM((bw,), jnp.int32),
            pltpu.VMEM((bw, D), jnp.float32),
            pltpu.VMEM((bw, D), jnp.float32),
            pltpu.SemaphoreType.DMA,
            pltpu.SemaphoreType.DMA,
            pltpu.SemaphoreType.DMA,
            pltpu.SemaphoreType.DMA,
        ],
    )
    def k(idxT_hbm, table_hbm, outT_hbm, ia, ib, ba, bb, gsa, gsb, wsa, wsb):
        wid = lax.axis_index("s") * _NC + lax.axis_index("c")
        b0 = wid * bw

        def stage(s, idx_v, buf, gsem, wsem):
            pltpu.sync_copy(idxT_hbm.at[s, pl.ds(b0, bw)], idx_v)
            pltpu.async_copy(table_hbm.at[idx_v], buf, gsem).wait()
            return pltpu.async_copy(buf, outT_hbm.at[s, pl.ds(b0, bw)], wsem)

        wa = stage(0, ia, ba, gsa, wsa)
        wb = stage(1, ib, bb, gsb, wsb)

        def body(i, carry):
            s = 2 * i
            wa.wait()
            stage(s, ia, ba, gsa, wsa)
            wb.wait()
            stage(s + 1, ib, bb, gsb, wsb)
            return carry

        lax.fori_loop(1, B1 // 2, body, 0)
        wa.wait()
        wb.wait()

    return k


def kernel(input, table):
    B0, B1 = input.shape
    V, D = table.shape
    outT = _build(B0, B1, V, D)(input.T.astype(jnp.int32), table)
    return outT.transpose(1, 0, 2)
